# baseline (speedup 1.0000x reference)
import functools

import jax
import jax.numpy as jnp
from jax import lax
from jax.experimental import pallas as pl
from jax.experimental.pallas import tpu as pltpu
from jax.experimental.pallas import tpu_sc as plsc

_EPS = 1e-5

_N = 50000
_E = 800000
_H = 64
_HH = 32
_HQ = 16
_BLK = 2000
_GRID = _N // _BLK

_CW = 128
_NCHUNK = 6400
_EP = _NCHUNK * _CW
_DCPT = _NCHUNK // 32
_CW2 = 128
_NCHUNK2 = _EP // _CW2
_CPT = _NCHUNK2 // 16
_SUPC = 40
_SUPS = _CPT // _SUPC
_ND = 50048
_NR = _ND
_TRASH = 50000
_RPT = _ND // 16
_ZR = _NR // 16


def _dotT(a, w):
  return lax.dot_general(a, w, (((1,), (1,)), ((), ())),
                         preferred_element_type=jnp.float32)


def _deg_body(dst2d, ones_h, zeros_h, out0, out1, acc, dall, ones_v):
  c = lax.axis_index("c")
  s = lax.axis_index("s")
  wid = c * 16 + s
  pltpu.sync_copy(ones_h, ones_v)
  pltpu.sync_copy(dst2d.at[pl.ds(wid * _DCPT, _DCPT)], dall)
  pltpu.sync_copy(zeros_h, acc.at[pl.ds(s * _ZR, _ZR)])
  plsc.subcore_barrier()

  @pl.loop(0, _DCPT)
  def _(j):
    pltpu.sync_copy(ones_v, acc.at[dall.at[j]], add=True)

  plsc.subcore_barrier()

  @pl.when(c == 0)
  def _():
    pltpu.sync_copy(acc.at[pl.ds(s * _RPT, _RPT)],
                    out0.at[pl.ds(s * _RPT, _RPT)])

  @pl.when(c == 1)
  def _():
    pltpu.sync_copy(acc.at[pl.ds(s * _RPT, _RPT)],
                    out1.at[pl.ds(s * _RPT, _RPT)])


def _deg_counts(dst2d, ones_h, zeros_h):
  mesh = plsc.VectorSubcoreMesh(core_axis_name="c", subcore_axis_name="s")
  f = pl.kernel(
      _deg_body,
      compiler_params=pltpu.CompilerParams(use_tc_tiling_on_sc=False),
      out_type=[jax.ShapeDtypeStruct((_ND, 8), jnp.float32),
                jax.ShapeDtypeStruct((_ND, 8), jnp.float32)],
      mesh=mesh,
      scratch_types=[
          pltpu.VMEM_SHARED((_NR, 8), jnp.float32),
          pltpu.VMEM((_DCPT, _CW), jnp.int32),
          pltpu.VMEM((_CW, 8), jnp.float32),
      ],
  )
  return f(dst2d, ones_h, zeros_h)


def _spmm_body(tA, tB, src2d, dst2d, oA, oB,
               tblS, accS, sall, dall, rowa, rowb, gsa, gsb, ssa, ssb):
  s = lax.axis_index("s")

  def run(tbl, out):
    rb = s * _RPT
    pltpu.sync_copy(tbl.at[pl.ds(rb, _RPT)], tblS.at[pl.ds(rb, _RPT)])
    pltpu.sync_copy(tbl.at[pl.ds(rb, _RPT)], accS.at[pl.ds(rb, _RPT)])
    plsc.subcore_barrier()

    @pl.loop(0, _SUPS)
    def _(u):
      base = s * _CPT + u * _SUPC
      pltpu.sync_copy(src2d.at[pl.ds(base, _SUPC)], sall)
      pltpu.sync_copy(dst2d.at[pl.ds(base, _SUPC)], dall)
      pltpu.async_copy(tblS.at[sall.at[0]], rowa, gsa)

      @pl.loop(0, _SUPC, step=2)
      def _(j):
        pltpu.make_async_copy(tblS.at[sall.at[j]], rowa, gsa).wait()

        @pl.when(j > 0)
        def _():
          pltpu.make_async_copy(rowb, accS.at[dall.at[j]], ssb).wait()

        pltpu.async_copy(tblS.at[sall.at[j + 1]], rowb, gsb)
        pltpu.async_copy(rowa, accS.at[dall.at[j]], ssa, add=True)
        pltpu.make_async_copy(tblS.at[sall.at[j + 1]], rowb, gsb).wait()
        pltpu.make_async_copy(rowa, accS.at[dall.at[j]], ssa).wait()

        @pl.when(j + 2 < _SUPC)
        def _():
          pltpu.async_copy(tblS.at[sall.at[j + 2]], rowa, gsa)

        pltpu.async_copy(rowb, accS.at[dall.at[j + 1]], ssb, add=True)

      pltpu.make_async_copy(rowb, accS.at[dall.at[0]], ssb).wait()

    plsc.subcore_barrier()
    pltpu.sync_copy(accS.at[pl.ds(rb, _RPT)], out.at[pl.ds(rb, _RPT)])

  run(tA, oA)
  run(tB, oB)


def _spmm_half(tA, tB, src2w, dst2w):
  mesh = plsc.VectorSubcoreMesh(core_axis_name="c", subcore_axis_name="s",
                                num_cores=1)
  qt = jax.ShapeDtypeStruct((_ND, _HQ), jnp.float32)
  f = pl.kernel(
      _spmm_body,
      compiler_params=pltpu.CompilerParams(use_tc_tiling_on_sc=False),
      out_type=[qt, qt],
      mesh=mesh,
      scratch_types=[
          pltpu.VMEM_SHARED((_NR, _HQ), jnp.float32),
          pltpu.VMEM_SHARED((_NR, _HQ), jnp.float32),
          pltpu.VMEM((_SUPC, _CW2), jnp.int32),
          pltpu.VMEM((_SUPC, _CW2), jnp.int32),
          pltpu.VMEM((_CW2, _HQ), jnp.float32),
          pltpu.VMEM((_CW2, _HQ), jnp.float32),
          pltpu.SemaphoreType.DMA,
          pltpu.SemaphoreType.DMA,
          pltpu.SemaphoreType.DMA,
          pltpu.SemaphoreType.DMA,
      ],
  )
  return f(tA, tB, src2w, dst2w)


def _spmm(t0, t1, t2, t3, src2w, dst2w):
  o0, o2 = _spmm_half(t0, t2, src2w, dst2w)
  o1, o3 = _spmm_half(t1, t3, src2w, dst2w)
  return o0, o1, o2, o3


def _proj_body(x_ref, d0_ref, d1_ref, wp_ref, bp_ref, wg_ref,
               g0_ref, g1_ref, g2_ref, g3_ref, dinv_ref):
  deg = d0_ref[:, 0:1] + d1_ref[:, 0:1] + 1.0
  dinv = lax.rsqrt(deg)
  h = jnp.maximum(_dotT(x_ref[...], wp_ref[...]) + bp_ref[...], 0.0)
  g = _dotT(h, wg_ref[...]) * dinv
  g0_ref[...] = g[:, 0 * _HQ:1 * _HQ]
  g1_ref[...] = g[:, 1 * _HQ:2 * _HQ]
  g2_ref[...] = g[:, 2 * _HQ:3 * _HQ]
  g3_ref[...] = g[:, 3 * _HQ:4 * _HQ]
  dinv_ref[...] = jnp.broadcast_to(dinv, dinv_ref.shape)


def _proj(x, d0, d1, wp, bp, wg):
  d = x.shape[1]
  return pl.pallas_call(
      _proj_body,
      grid=(_GRID,),
      in_specs=[
          pl.BlockSpec((_BLK, d), lambda i: (i, 0)),
          pl.BlockSpec((_BLK, 8), lambda i: (i, 0)),
          pl.BlockSpec((_BLK, 8), lambda i: (i, 0)),
          pl.BlockSpec((_H, d), lambda i: (0, 0)),
          pl.BlockSpec((1, _H), lambda i: (0, 0)),
          pl.BlockSpec((_H, _H), lambda i: (0, 0)),
      ],
      out_specs=[
          pl.BlockSpec((_BLK, _HQ), lambda i: (i, 0)),
          pl.BlockSpec((_BLK, _HQ), lambda i: (i, 0)),
          pl.BlockSpec((_BLK, _HQ), lambda i: (i, 0)),
          pl.BlockSpec((_BLK, _HQ), lambda i: (i, 0)),
          pl.BlockSpec((_BLK, 8), lambda i: (i, 0)),
      ],
      out_shape=[
          jax.ShapeDtypeStruct((_ND, _HQ), jnp.float32),
          jax.ShapeDtypeStruct((_ND, _HQ), jnp.float32),
          jax.ShapeDtypeStruct((_ND, _HQ), jnp.float32),
          jax.ShapeDtypeStruct((_ND, _HQ), jnp.float32),
          jax.ShapeDtypeStruct((_N, 8), jnp.float32),
      ],
  )(x, d0, d1, wp, bp, wg)


def _stats_body(a0_ref, a1_ref, a2_ref, a3_ref, dinv_ref, bg_ref, s_ref):
  @pl.when(pl.program_id(0) == 0)
  def _():
    s_ref[...] = jnp.zeros_like(s_ref)

  dinv = dinv_ref[:, 0:1]
  y = jnp.concatenate(
      [a0_ref[...], a1_ref[...], a2_ref[...], a3_ref[...]],
      axis=1) * dinv + bg_ref[...]
  s1 = jnp.sum(y, axis=0, keepdims=True)
  s2 = jnp.sum(y * y, axis=0, keepdims=True)
  s_ref[...] += jnp.concatenate([s1, s2, jnp.zeros((6, _H), jnp.float32)], 0)


def _stats(a0, a1, a2, a3, dinv, bg):
  return pl.pallas_call(
      _stats_body,
      grid=(_GRID,),
      in_specs=[
          pl.BlockSpec((_BLK, _HQ), lambda i: (i, 0)),
          pl.BlockSpec((_BLK, _HQ), lambda i: (i, 0)),
          pl.BlockSpec((_BLK, _HQ), lambda i: (i, 0)),
          pl.BlockSpec((_BLK, _HQ), lambda i: (i, 0)),
          pl.BlockSpec((_BLK, 8), lambda i: (i, 0)),
          pl.BlockSpec((1, _H), lambda i: (0, 0)),
      ],
      out_specs=pl.BlockSpec((8, _H), lambda i: (0, 0)),
      out_shape=jax.ShapeDtypeStruct((8, _H), jnp.float32),
  )(a0, a1, a2, a3, dinv, bg)


def _bn_from_sums(s_ref, gam_ref, bet_ref):
  m = s_ref[0:1, :] * (1.0 / _N)
  var = s_ref[1:2, :] * (1.0 / _N) - m * m
  scale = lax.rsqrt(var + _EPS) * gam_ref[...]
  shift = bet_ref[...] - m * scale
  return scale, shift


def _mid_body(a0_ref, a1_ref, a2_ref, a3_ref, dinv_ref, s_ref, bg_ref,
              gam_ref, bet_ref, wn_ref, g0_ref, g1_ref, g2_ref, g3_ref):
  dinv = dinv_ref[:, 0:1]
  scale, shift = _bn_from_sums(s_ref, gam_ref, bet_ref)
  y = jnp.concatenate(
      [a0_ref[...], a1_ref[...], a2_ref[...], a3_ref[...]],
      axis=1) * dinv + bg_ref[...]
  h = jnp.maximum(y * scale + shift, 0.0)
  g = _dotT(h, wn_ref[...]) * dinv
  g0_ref[...] = g[:, 0 * _HQ:1 * _HQ]
  g1_ref[...] = g[:, 1 * _HQ:2 * _HQ]
  g2_ref[...] = g[:, 2 * _HQ:3 * _HQ]
  g3_ref[...] = g[:, 3 * _HQ:4 * _HQ]


def _mid(a0, a1, a2, a3, dinv, s, bg, gam, bet, wn):
  return pl.pallas_call(
      _mid_body,
      grid=(_GRID,),
      in_specs=[
          pl.BlockSpec((_BLK, _HQ), lambda i: (i, 0)),
          pl.BlockSpec((_BLK, _HQ), lambda i: (i, 0)),
          pl.BlockSpec((_BLK, _HQ), lambda i: (i, 0)),
          pl.BlockSpec((_BLK, _HQ), lambda i: (i, 0)),
          pl.BlockSpec((_BLK, 8), lambda i: (i, 0)),
          pl.BlockSpec((8, _H), lambda i: (0, 0)),
          pl.BlockSpec((1, _H), lambda i: (0, 0)),
          pl.BlockSpec((1, _H), lambda i: (0, 0)),
          pl.BlockSpec((1, _H), lambda i: (0, 0)),
          pl.BlockSpec((_H, _H), lambda i: (0, 0)),
      ],
      out_specs=[
          pl.BlockSpec((_BLK, _HQ), lambda i: (i, 0)),
          pl.BlockSpec((_BLK, _HQ), lambda i: (i, 0)),
          pl.BlockSpec((_BLK, _HQ), lambda i: (i, 0)),
          pl.BlockSpec((_BLK, _HQ), lambda i: (i, 0)),
      ],
      out_shape=[
          jax.ShapeDtypeStruct((_ND, _HQ), jnp.float32),
          jax.ShapeDtypeStruct((_ND, _HQ), jnp.float32),
          jax.ShapeDtypeStruct((_ND, _HQ), jnp.float32),
          jax.ShapeDtypeStruct((_ND, _HQ), jnp.float32),
      ],
  )(a0, a1, a2, a3, dinv, s, bg, gam, bet, wn)


def _fin_body(a0_ref, a1_ref, a2_ref, a3_ref, dinv_ref, s_ref, bg_ref,
              gam_ref, bet_ref, wc1_ref, bc1_ref, wc2_ref, bc2_ref, o_ref):
  dinv = dinv_ref[:, 0:1]
  scale, shift = _bn_from_sums(s_ref, gam_ref, bet_ref)
  y = jnp.concatenate(
      [a0_ref[...], a1_ref[...], a2_ref[...], a3_ref[...]],
      axis=1) * dinv + bg_ref[...]
  h = y * scale + shift
  t = jnp.maximum(_dotT(h, wc1_ref[...]) + bc1_ref[...], 0.0)
  o_ref[...] = _dotT(t, wc2_ref[...]) + bc2_ref[...]


def _fin(a0, a1, a2, a3, dinv, s, bg, gam, bet, wc1, bc1, wc2, bc2):
  hc = wc1.shape[0]
  nc = wc2.shape[0]
  return pl.pallas_call(
      _fin_body,
      grid=(_GRID,),
      in_specs=[
          pl.BlockSpec((_BLK, _HQ), lambda i: (i, 0)),
          pl.BlockSpec((_BLK, _HQ), lambda i: (i, 0)),
          pl.BlockSpec((_BLK, _HQ), lambda i: (i, 0)),
          pl.BlockSpec((_BLK, _HQ), lambda i: (i, 0)),
          pl.BlockSpec((_BLK, 8), lambda i: (i, 0)),
          pl.BlockSpec((8, _H), lambda i: (0, 0)),
          pl.BlockSpec((1, _H), lambda i: (0, 0)),
          pl.BlockSpec((1, _H), lambda i: (0, 0)),
          pl.BlockSpec((1, _H), lambda i: (0, 0)),
          pl.BlockSpec((hc, _H), lambda i: (0, 0)),
          pl.BlockSpec((1, hc), lambda i: (0, 0)),
          pl.BlockSpec((nc, hc), lambda i: (0, 0)),
          pl.BlockSpec((1, nc), lambda i: (0, 0)),
      ],
      out_specs=pl.BlockSpec((_BLK, nc), lambda i: (i, 0)),
      out_shape=jax.ShapeDtypeStruct((_N, nc), jnp.float32),
  )(a0, a1, a2, a3, dinv, s, bg, gam, bet, wc1, bc1, wc2, bc2)


def kernel(x, edge_index, Wp, bp, Wg0, bg0, Wg1, bg1, Wg2, bg2,
           gamma0, beta0, gamma1, beta1, gamma2, beta2,
           Wc1, bc1, Wc2, bc2):
  src = edge_index[0]
  dst = edge_index[1]
  pad = _EP - _E
  src_p = jnp.concatenate([src, jnp.zeros((pad,), jnp.int32)])
  dst_p = jnp.concatenate([dst, jnp.full((pad,), _TRASH, jnp.int32)])
  dst2d = dst_p.reshape(_NCHUNK, _CW)
  src2w = src_p.reshape(_NCHUNK2, _CW2)
  dst2w = dst_p.reshape(_NCHUNK2, _CW2)

  ones_h = jnp.ones((_CW, 8), jnp.float32)
  zeros_h = jnp.zeros((_ZR, 8), jnp.float32)

  bp_ = bp.reshape(1, _H)
  bg0_ = bg0.reshape(1, _H)
  bg1_ = bg1.reshape(1, _H)
  bg2_ = bg2.reshape(1, _H)
  ga0_ = gamma0.reshape(1, _H)
  be0_ = beta0.reshape(1, _H)
  ga1_ = gamma1.reshape(1, _H)
  be1_ = beta1.reshape(1, _H)
  ga2_ = gamma2.reshape(1, _H)
  be2_ = beta2.reshape(1, _H)
  bc1_ = bc1.reshape(1, -1)
  bc2_ = bc2.reshape(1, -1)

  d0, d1 = _deg_counts(dst2d, ones_h, zeros_h)

  t0, t1, t2, t3, dinv = _proj(x, d0, d1, Wp, bp_, Wg0)

  a0, a1, a2, a3 = _spmm(t0, t1, t2, t3, src2w, dst2w)
  s = _stats(a0, a1, a2, a3, dinv, bg0_)
  t0, t1, t2, t3 = _mid(a0, a1, a2, a3, dinv, s, bg0_, ga0_, be0_, Wg1)

  a0, a1, a2, a3 = _spmm(t0, t1, t2, t3, src2w, dst2w)
  s = _stats(a0, a1, a2, a3, dinv, bg1_)
  t0, t1, t2, t3 = _mid(a0, a1, a2, a3, dinv, s, bg1_, ga1_, be1_, Wg2)

  a0, a1, a2, a3 = _spmm(t0, t1, t2, t3, src2w, dst2w)
  s = _stats(a0, a1, a2, a3, dinv, bg2_)
  return _fin(a0, a1, a2, a3, dinv, s, bg2_, ga2_, be2_, Wc1, bc1_, Wc2, bc2_)


# speedup vs baseline: 1.3357x; 1.3357x over previous
import functools

import jax
import jax.numpy as jnp
from jax import lax
from jax.experimental import pallas as pl
from jax.experimental.pallas import tpu as pltpu
from jax.experimental.pallas import tpu_sc as plsc

_EPS = 1e-5

_N = 50000
_E = 800000
_H = 64
_HH = 32
_HQ = 16
_BLK = 5000
_GRID = _N // _BLK

_CW = 128
_NCHUNK = 6400
_EP = _NCHUNK * _CW
_CPT = _NCHUNK // 16
_DCPT = _NCHUNK // 32
_SUPC = 40
_SUPS = _CPT // _SUPC
_ND = 50048
_NR = _ND
_TRASH = 50000
_RPT = _ND // 16
_ZR = _NR // 16


def _dotT(a, w):
  return lax.dot_general(a, w, (((1,), (1,)), ((), ())),
                         preferred_element_type=jnp.float32)


def _deg_body(dst2d, ones_h, zeros_h, out0, out1, acc, dall, ones_v):
  c = lax.axis_index("c")
  s = lax.axis_index("s")
  wid = c * 16 + s
  pltpu.sync_copy(ones_h, ones_v)
  pltpu.sync_copy(dst2d.at[pl.ds(wid * _DCPT, _DCPT)], dall)
  pltpu.sync_copy(zeros_h, acc.at[pl.ds(s * _ZR, _ZR)])
  plsc.subcore_barrier()

  @pl.loop(0, _DCPT)
  def _(j):
    pltpu.sync_copy(ones_v, acc.at[dall.at[j]], add=True)

  plsc.subcore_barrier()

  @pl.when(c == 0)
  def _():
    pltpu.sync_copy(acc.at[pl.ds(s * _RPT, _RPT)],
                    out0.at[pl.ds(s * _RPT, _RPT)])

  @pl.when(c == 1)
  def _():
    pltpu.sync_copy(acc.at[pl.ds(s * _RPT, _RPT)],
                    out1.at[pl.ds(s * _RPT, _RPT)])


def _deg_counts(dst2d, ones_h, zeros_h):
  mesh = plsc.VectorSubcoreMesh(core_axis_name="c", subcore_axis_name="s")
  f = pl.kernel(
      _deg_body,
      compiler_params=pltpu.CompilerParams(use_tc_tiling_on_sc=False),
      out_type=[jax.ShapeDtypeStruct((_ND, 8), jnp.float32),
                jax.ShapeDtypeStruct((_ND, 8), jnp.float32)],
      mesh=mesh,
      scratch_types=[
          pltpu.VMEM_SHARED((_NR, 8), jnp.float32),
          pltpu.VMEM((_DCPT, _CW), jnp.int32),
          pltpu.VMEM((_CW, 8), jnp.float32),
      ],
  )
  return f(dst2d, ones_h, zeros_h)


def _spmm_body(t0, t1, t2, t3, src2d, dst2d, o0, o1, o2, o3,
               tblS, accS, sall, dall, rowa, rowb, gsa, gsb, ssa, ssb):
  c = lax.axis_index("c")
  s = lax.axis_index("s")

  def run(tbl, out):
    rb = s * _RPT
    pltpu.sync_copy(tbl.at[pl.ds(rb, _RPT)], tblS.at[pl.ds(rb, _RPT)])
    pltpu.sync_copy(tbl.at[pl.ds(rb, _RPT)], accS.at[pl.ds(rb, _RPT)])
    plsc.subcore_barrier()

    @pl.loop(0, _SUPS)
    def _(u):
      base = s * _CPT + u * _SUPC
      pltpu.sync_copy(src2d.at[pl.ds(base, _SUPC)], sall)
      pltpu.sync_copy(dst2d.at[pl.ds(base, _SUPC)], dall)
      pltpu.async_copy(tblS.at[sall.at[0]], rowa, gsa)

      @pl.loop(0, _SUPC, step=2)
      def _(j):
        pltpu.make_async_copy(tblS.at[sall.at[j]], rowa, gsa).wait()

        @pl.when(j > 0)
        def _():
          pltpu.make_async_copy(rowb, accS.at[dall.at[j]], ssb).wait()

        pltpu.async_copy(tblS.at[sall.at[j + 1]], rowb, gsb)
        pltpu.async_copy(rowa, accS.at[dall.at[j]], ssa, add=True)
        pltpu.make_async_copy(tblS.at[sall.at[j + 1]], rowb, gsb).wait()
        pltpu.make_async_copy(rowa, accS.at[dall.at[j]], ssa).wait()

        @pl.when(j + 2 < _SUPC)
        def _():
          pltpu.async_copy(tblS.at[sall.at[j + 2]], rowa, gsa)

        pltpu.async_copy(rowb, accS.at[dall.at[j + 1]], ssb, add=True)

      pltpu.make_async_copy(rowb, accS.at[dall.at[0]], ssb).wait()

    plsc.subcore_barrier()
    pltpu.sync_copy(accS.at[pl.ds(rb, _RPT)], out.at[pl.ds(rb, _RPT)])

  @pl.when(c == 0)
  def _():
    run(t0, o0)
    run(t2, o2)

  @pl.when(c == 1)
  def _():
    run(t1, o1)
    run(t3, o3)


def _spmm(t0, t1, t2, t3, src2d, dst2d):
  mesh = plsc.VectorSubcoreMesh(core_axis_name="c", subcore_axis_name="s")
  qt = jax.ShapeDtypeStruct((_ND, _HQ), jnp.float32)
  f = pl.kernel(
      _spmm_body,
      compiler_params=pltpu.CompilerParams(use_tc_tiling_on_sc=False),
      out_type=[qt, qt, qt, qt],
      mesh=mesh,
      scratch_types=[
          pltpu.VMEM_SHARED((_NR, _HQ), jnp.float32),
          pltpu.VMEM_SHARED((_NR, _HQ), jnp.float32),
          pltpu.VMEM((_SUPC, _CW), jnp.int32),
          pltpu.VMEM((_SUPC, _CW), jnp.int32),
          pltpu.VMEM((_CW, _HQ), jnp.float32),
          pltpu.VMEM((_CW, _HQ), jnp.float32),
          pltpu.SemaphoreType.DMA,
          pltpu.SemaphoreType.DMA,
          pltpu.SemaphoreType.DMA,
          pltpu.SemaphoreType.DMA,
      ],
  )
  return f(t0, t1, t2, t3, src2d, dst2d)


def _proj_body(x_ref, d0_ref, d1_ref, wp_ref, bp_ref, wg_ref,
               g0_ref, g1_ref, g2_ref, g3_ref, dinv_ref):
  deg = d0_ref[:, 0:1] + d1_ref[:, 0:1] + 1.0
  dinv = lax.rsqrt(deg)
  h = jnp.maximum(_dotT(x_ref[...], wp_ref[...]) + bp_ref[...], 0.0)
  g = _dotT(h, wg_ref[...]) * dinv
  g0_ref[...] = g[:, 0 * _HQ:1 * _HQ]
  g1_ref[...] = g[:, 1 * _HQ:2 * _HQ]
  g2_ref[...] = g[:, 2 * _HQ:3 * _HQ]
  g3_ref[...] = g[:, 3 * _HQ:4 * _HQ]
  dinv_ref[...] = jnp.broadcast_to(dinv, dinv_ref.shape)


def _proj(x, d0, d1, wp, bp, wg):
  d = x.shape[1]
  return pl.pallas_call(
      _proj_body,
      grid=(_GRID,),
      in_specs=[
          pl.BlockSpec((_BLK, d), lambda i: (i, 0)),
          pl.BlockSpec((_BLK, 8), lambda i: (i, 0)),
          pl.BlockSpec((_BLK, 8), lambda i: (i, 0)),
          pl.BlockSpec((_H, d), lambda i: (0, 0)),
          pl.BlockSpec((1, _H), lambda i: (0, 0)),
          pl.BlockSpec((_H, _H), lambda i: (0, 0)),
      ],
      out_specs=[
          pl.BlockSpec((_BLK, _HQ), lambda i: (i, 0)),
          pl.BlockSpec((_BLK, _HQ), lambda i: (i, 0)),
          pl.BlockSpec((_BLK, _HQ), lambda i: (i, 0)),
          pl.BlockSpec((_BLK, _HQ), lambda i: (i, 0)),
          pl.BlockSpec((_BLK, 8), lambda i: (i, 0)),
      ],
      out_shape=[
          jax.ShapeDtypeStruct((_ND, _HQ), jnp.float32),
          jax.ShapeDtypeStruct((_ND, _HQ), jnp.float32),
          jax.ShapeDtypeStruct((_ND, _HQ), jnp.float32),
          jax.ShapeDtypeStruct((_ND, _HQ), jnp.float32),
          jax.ShapeDtypeStruct((_N, 8), jnp.float32),
      ],
  )(x, d0, d1, wp, bp, wg)


def _stats_body(a0_ref, a1_ref, a2_ref, a3_ref, dinv_ref, bg_ref, s_ref):
  @pl.when(pl.program_id(0) == 0)
  def _():
    s_ref[...] = jnp.zeros_like(s_ref)

  dinv = dinv_ref[:, 0:1]
  y = jnp.concatenate(
      [a0_ref[...], a1_ref[...], a2_ref[...], a3_ref[...]],
      axis=1) * dinv + bg_ref[...]
  s1 = jnp.sum(y, axis=0, keepdims=True)
  s2 = jnp.sum(y * y, axis=0, keepdims=True)
  s_ref[...] += jnp.concatenate([s1, s2, jnp.zeros((6, _H), jnp.float32)], 0)


def _stats(a0, a1, a2, a3, dinv, bg):
  return pl.pallas_call(
      _stats_body,
      grid=(_GRID,),
      in_specs=[
          pl.BlockSpec((_BLK, _HQ), lambda i: (i, 0)),
          pl.BlockSpec((_BLK, _HQ), lambda i: (i, 0)),
          pl.BlockSpec((_BLK, _HQ), lambda i: (i, 0)),
          pl.BlockSpec((_BLK, _HQ), lambda i: (i, 0)),
          pl.BlockSpec((_BLK, 8), lambda i: (i, 0)),
          pl.BlockSpec((1, _H), lambda i: (0, 0)),
      ],
      out_specs=pl.BlockSpec((8, _H), lambda i: (0, 0)),
      out_shape=jax.ShapeDtypeStruct((8, _H), jnp.float32),
  )(a0, a1, a2, a3, dinv, bg)


def _bn_from_sums(s_ref, gam_ref, bet_ref):
  m = s_ref[0:1, :] * (1.0 / _N)
  var = s_ref[1:2, :] * (1.0 / _N) - m * m
  scale = lax.rsqrt(var + _EPS) * gam_ref[...]
  shift = bet_ref[...] - m * scale
  return scale, shift


def _mid_body(a0_ref, a1_ref, a2_ref, a3_ref, dinv_ref, s_ref, bg_ref,
              gam_ref, bet_ref, wn_ref, g0_ref, g1_ref, g2_ref, g3_ref):
  dinv = dinv_ref[:, 0:1]
  scale, shift = _bn_from_sums(s_ref, gam_ref, bet_ref)
  y = jnp.concatenate(
      [a0_ref[...], a1_ref[...], a2_ref[...], a3_ref[...]],
      axis=1) * dinv + bg_ref[...]
  h = jnp.maximum(y * scale + shift, 0.0)
  g = _dotT(h, wn_ref[...]) * dinv
  g0_ref[...] = g[:, 0 * _HQ:1 * _HQ]
  g1_ref[...] = g[:, 1 * _HQ:2 * _HQ]
  g2_ref[...] = g[:, 2 * _HQ:3 * _HQ]
  g3_ref[...] = g[:, 3 * _HQ:4 * _HQ]


def _mid(a0, a1, a2, a3, dinv, s, bg, gam, bet, wn):
  return pl.pallas_call(
      _mid_body,
      grid=(_GRID,),
      in_specs=[
          pl.BlockSpec((_BLK, _HQ), lambda i: (i, 0)),
          pl.BlockSpec((_BLK, _HQ), lambda i: (i, 0)),
          pl.BlockSpec((_BLK, _HQ), lambda i: (i, 0)),
          pl.BlockSpec((_BLK, _HQ), lambda i: (i, 0)),
          pl.BlockSpec((_BLK, 8), lambda i: (i, 0)),
          pl.BlockSpec((8, _H), lambda i: (0, 0)),
          pl.BlockSpec((1, _H), lambda i: (0, 0)),
          pl.BlockSpec((1, _H), lambda i: (0, 0)),
          pl.BlockSpec((1, _H), lambda i: (0, 0)),
          pl.BlockSpec((_H, _H), lambda i: (0, 0)),
      ],
      out_specs=[
          pl.BlockSpec((_BLK, _HQ), lambda i: (i, 0)),
          pl.BlockSpec((_BLK, _HQ), lambda i: (i, 0)),
          pl.BlockSpec((_BLK, _HQ), lambda i: (i, 0)),
          pl.BlockSpec((_BLK, _HQ), lambda i: (i, 0)),
      ],
      out_shape=[
          jax.ShapeDtypeStruct((_ND, _HQ), jnp.float32),
          jax.ShapeDtypeStruct((_ND, _HQ), jnp.float32),
          jax.ShapeDtypeStruct((_ND, _HQ), jnp.float32),
          jax.ShapeDtypeStruct((_ND, _HQ), jnp.float32),
      ],
  )(a0, a1, a2, a3, dinv, s, bg, gam, bet, wn)


def _fin_body(a0_ref, a1_ref, a2_ref, a3_ref, dinv_ref, s_ref, bg_ref,
              gam_ref, bet_ref, wc1_ref, bc1_ref, wc2_ref, bc2_ref, o_ref):
  dinv = dinv_ref[:, 0:1]
  scale, shift = _bn_from_sums(s_ref, gam_ref, bet_ref)
  y = jnp.concatenate(
      [a0_ref[...], a1_ref[...], a2_ref[...], a3_ref[...]],
      axis=1) * dinv + bg_ref[...]
  h = y * scale + shift
  t = jnp.maximum(_dotT(h, wc1_ref[...]) + bc1_ref[...], 0.0)
  o_ref[...] = _dotT(t, wc2_ref[...]) + bc2_ref[...]


def _fin(a0, a1, a2, a3, dinv, s, bg, gam, bet, wc1, bc1, wc2, bc2):
  hc = wc1.shape[0]
  nc = wc2.shape[0]
  return pl.pallas_call(
      _fin_body,
      grid=(_GRID,),
      in_specs=[
          pl.BlockSpec((_BLK, _HQ), lambda i: (i, 0)),
          pl.BlockSpec((_BLK, _HQ), lambda i: (i, 0)),
          pl.BlockSpec((_BLK, _HQ), lambda i: (i, 0)),
          pl.BlockSpec((_BLK, _HQ), lambda i: (i, 0)),
          pl.BlockSpec((_BLK, 8), lambda i: (i, 0)),
          pl.BlockSpec((8, _H), lambda i: (0, 0)),
          pl.BlockSpec((1, _H), lambda i: (0, 0)),
          pl.BlockSpec((1, _H), lambda i: (0, 0)),
          pl.BlockSpec((1, _H), lambda i: (0, 0)),
          pl.BlockSpec((hc, _H), lambda i: (0, 0)),
          pl.BlockSpec((1, hc), lambda i: (0, 0)),
          pl.BlockSpec((nc, hc), lambda i: (0, 0)),
          pl.BlockSpec((1, nc), lambda i: (0, 0)),
      ],
      out_specs=pl.BlockSpec((_BLK, nc), lambda i: (i, 0)),
      out_shape=jax.ShapeDtypeStruct((_N, nc), jnp.float32),
  )(a0, a1, a2, a3, dinv, s, bg, gam, bet, wc1, bc1, wc2, bc2)


def kernel(x, edge_index, Wp, bp, Wg0, bg0, Wg1, bg1, Wg2, bg2,
           gamma0, beta0, gamma1, beta1, gamma2, beta2,
           Wc1, bc1, Wc2, bc2):
  src = edge_index[0]
  dst = edge_index[1]
  pad = _EP - _E
  src2d = jnp.concatenate(
      [src, jnp.zeros((pad,), jnp.int32)]).reshape(_NCHUNK, _CW)
  dst2d = jnp.concatenate(
      [dst, jnp.full((pad,), _TRASH, jnp.int32)]).reshape(_NCHUNK, _CW)

  ones_h = jnp.ones((_CW, 8), jnp.float32)
  zeros_h = jnp.zeros((_ZR, 8), jnp.float32)

  bp_ = bp.reshape(1, _H)
  bg0_ = bg0.reshape(1, _H)
  bg1_ = bg1.reshape(1, _H)
  bg2_ = bg2.reshape(1, _H)
  ga0_ = gamma0.reshape(1, _H)
  be0_ = beta0.reshape(1, _H)
  ga1_ = gamma1.reshape(1, _H)
  be1_ = beta1.reshape(1, _H)
  ga2_ = gamma2.reshape(1, _H)
  be2_ = beta2.reshape(1, _H)
  bc1_ = bc1.reshape(1, -1)
  bc2_ = bc2.reshape(1, -1)

  d0, d1 = _deg_counts(dst2d, ones_h, zeros_h)

  t0, t1, t2, t3, dinv = _proj(x, d0, d1, Wp, bp_, Wg0)

  a0, a1, a2, a3 = _spmm(t0, t1, t2, t3, src2d, dst2d)
  s = _stats(a0, a1, a2, a3, dinv, bg0_)
  t0, t1, t2, t3 = _mid(a0, a1, a2, a3, dinv, s, bg0_, ga0_, be0_, Wg1)

  a0, a1, a2, a3 = _spmm(t0, t1, t2, t3, src2d, dst2d)
  s = _stats(a0, a1, a2, a3, dinv, bg1_)
  t0, t1, t2, t3 = _mid(a0, a1, a2, a3, dinv, s, bg1_, ga1_, be1_, Wg2)

  a0, a1, a2, a3 = _spmm(t0, t1, t2, t3, src2d, dst2d)
  s = _stats(a0, a1, a2, a3, dinv, bg2_)
  return _fin(a0, a1, a2, a3, dinv, s, bg2_, ga2_, be2_, Wc1, bc1_, Wc2, bc2_)


# speedup vs baseline: 1.5847x; 1.1864x over previous
import functools

import jax
import jax.numpy as jnp
from jax import lax
from jax.experimental import pallas as pl
from jax.experimental.pallas import tpu as pltpu
from jax.experimental.pallas import tpu_sc as plsc

_EPS = 1e-5

_N = 50000
_E = 800000
_H = 64
_HH = 32
_HQ = 16
_BLK = 5000
_GRID = _N // _BLK

_CW = 128
_NCHUNK = 6400
_EP = _NCHUNK * _CW
_CPT = _NCHUNK // 16
_DCPT = _NCHUNK // 32
_SUPC = 40
_SUPS = _CPT // _SUPC
_ND = 50048
_NR = _ND
_TRASH = 50000
_RPT = _ND // 16
_ZR = _NR // 16


def _dotT(a, w):
  return lax.dot_general(a, w, (((1,), (1,)), ((), ())),
                         preferred_element_type=jnp.float32)


def _deg_body(dst2d, ones_h, zeros_h, out0, out1, acc, dall, ones_v):
  c = lax.axis_index("c")
  s = lax.axis_index("s")
  wid = c * 16 + s
  pltpu.sync_copy(ones_h, ones_v)
  pltpu.sync_copy(dst2d.at[pl.ds(wid * _DCPT, _DCPT)], dall)
  pltpu.sync_copy(zeros_h, acc.at[pl.ds(s * _ZR, _ZR)])
  plsc.subcore_barrier()

  @pl.loop(0, _DCPT)
  def _(j):
    pltpu.sync_copy(ones_v, acc.at[dall.at[j]], add=True)

  plsc.subcore_barrier()

  @pl.when(c == 0)
  def _():
    pltpu.sync_copy(acc.at[pl.ds(s * _RPT, _RPT)],
                    out0.at[pl.ds(s * _RPT, _RPT)])

  @pl.when(c == 1)
  def _():
    pltpu.sync_copy(acc.at[pl.ds(s * _RPT, _RPT)],
                    out1.at[pl.ds(s * _RPT, _RPT)])


def _deg_counts(dst2d, ones_h, zeros_h):
  mesh = plsc.VectorSubcoreMesh(core_axis_name="c", subcore_axis_name="s")
  f = pl.kernel(
      _deg_body,
      compiler_params=pltpu.CompilerParams(use_tc_tiling_on_sc=False),
      out_type=[jax.ShapeDtypeStruct((_ND, 8), jnp.float32),
                jax.ShapeDtypeStruct((_ND, 8), jnp.float32)],
      mesh=mesh,
      scratch_types=[
          pltpu.VMEM_SHARED((_NR, 8), jnp.float32),
          pltpu.VMEM((_DCPT, _CW), jnp.int32),
          pltpu.VMEM((_CW, 8), jnp.float32),
      ],
  )
  return f(dst2d, ones_h, zeros_h)


def _spmm_body(tcat, src2d, dst2d, ocat, tblS, accS, sall, dall,
               rowa, rowb, gsa, gsb, ssa, ssb):
  c = lax.axis_index("c")
  s = lax.axis_index("s")

  def run(q):
    rb = s * _RPT
    cb = pl.multiple_of(16 * (2 * c + q), 16)
    pltpu.sync_copy(tcat.at[pl.ds(rb, _RPT), pl.ds(cb, _HQ)],
                    tblS.at[pl.ds(rb, _RPT)])
    pltpu.sync_copy(tcat.at[pl.ds(rb, _RPT), pl.ds(cb, _HQ)],
                    accS.at[pl.ds(rb, _RPT)])
    plsc.subcore_barrier()

    @pl.loop(0, _SUPS)
    def _(u):
      base = s * _CPT + u * _SUPC
      pltpu.sync_copy(src2d.at[pl.ds(base, _SUPC)], sall)
      pltpu.sync_copy(dst2d.at[pl.ds(base, _SUPC)], dall)
      pltpu.async_copy(tblS.at[sall.at[0]], rowa, gsa)

      @pl.loop(0, _SUPC, step=2)
      def _(j):
        pltpu.make_async_copy(tblS.at[sall.at[j]], rowa, gsa).wait()

        @pl.when(j > 0)
        def _():
          pltpu.make_async_copy(rowb, accS.at[dall.at[j]], ssb).wait()

        pltpu.async_copy(tblS.at[sall.at[j + 1]], rowb, gsb)
        pltpu.async_copy(rowa, accS.at[dall.at[j]], ssa, add=True)
        pltpu.make_async_copy(tblS.at[sall.at[j + 1]], rowb, gsb).wait()
        pltpu.make_async_copy(rowa, accS.at[dall.at[j]], ssa).wait()

        @pl.when(j + 2 < _SUPC)
        def _():
          pltpu.async_copy(tblS.at[sall.at[j + 2]], rowa, gsa)

        pltpu.async_copy(rowb, accS.at[dall.at[j + 1]], ssb, add=True)

      pltpu.make_async_copy(rowb, accS.at[dall.at[0]], ssb).wait()

    plsc.subcore_barrier()
    pltpu.sync_copy(accS.at[pl.ds(rb, _RPT)],
                    ocat.at[pl.ds(rb, _RPT), pl.ds(cb, _HQ)])

  run(0)
  run(1)


def _spmm(tcat, src2w, dst2w):
  mesh = plsc.VectorSubcoreMesh(core_axis_name="c", subcore_axis_name="s")
  f = pl.kernel(
      _spmm_body,
      compiler_params=pltpu.CompilerParams(use_tc_tiling_on_sc=False),
      out_type=jax.ShapeDtypeStruct((_ND, _H), jnp.float32),
      mesh=mesh,
      scratch_types=[
          pltpu.VMEM_SHARED((_NR, _HQ), jnp.float32),
          pltpu.VMEM_SHARED((_NR, _HQ), jnp.float32),
          pltpu.VMEM((_SUPC, _CW), jnp.int32),
          pltpu.VMEM((_SUPC, _CW), jnp.int32),
          pltpu.VMEM((_CW, _HQ), jnp.float32),
          pltpu.VMEM((_CW, _HQ), jnp.float32),
          pltpu.SemaphoreType.DMA,
          pltpu.SemaphoreType.DMA,
          pltpu.SemaphoreType.DMA,
          pltpu.SemaphoreType.DMA,
      ],
  )
  return f(tcat, src2w, dst2w)


def _proj_body(x_ref, d0_ref, d1_ref, wp_ref, bp_ref, wg_ref,
               g_ref, dinv_ref):
  deg = d0_ref[:, 0:1] + d1_ref[:, 0:1] + 1.0
  dinv = lax.rsqrt(deg)
  h = jnp.maximum(_dotT(x_ref[...], wp_ref[...]) + bp_ref[...], 0.0)
  g_ref[...] = _dotT(h, wg_ref[...]) * dinv
  dinv_ref[...] = jnp.broadcast_to(dinv, dinv_ref.shape)


def _proj(x, d0, d1, wp, bp, wg):
  d = x.shape[1]
  return pl.pallas_call(
      _proj_body,
      grid=(_GRID,),
      in_specs=[
          pl.BlockSpec((_BLK, d), lambda i: (i, 0)),
          pl.BlockSpec((_BLK, 8), lambda i: (i, 0)),
          pl.BlockSpec((_BLK, 8), lambda i: (i, 0)),
          pl.BlockSpec((_H, d), lambda i: (0, 0)),
          pl.BlockSpec((1, _H), lambda i: (0, 0)),
          pl.BlockSpec((_H, _H), lambda i: (0, 0)),
      ],
      out_specs=[
          pl.BlockSpec((_BLK, _H), lambda i: (i, 0)),
          pl.BlockSpec((_BLK, 8), lambda i: (i, 0)),
      ],
      out_shape=[
          jax.ShapeDtypeStruct((_ND, _H), jnp.float32),
          jax.ShapeDtypeStruct((_N, 8), jnp.float32),
      ],
  )(x, d0, d1, wp, bp, wg)


def _stats_body(a_ref, dinv_ref, bg_ref, s_ref):
  @pl.when(pl.program_id(0) == 0)
  def _():
    s_ref[...] = jnp.zeros_like(s_ref)

  dinv = dinv_ref[:, 0:1]
  y = a_ref[...] * dinv + bg_ref[...]
  s1 = jnp.sum(y, axis=0, keepdims=True)
  s2 = jnp.sum(y * y, axis=0, keepdims=True)
  s_ref[...] += jnp.concatenate([s1, s2, jnp.zeros((6, _H), jnp.float32)], 0)


def _stats(a, dinv, bg):
  return pl.pallas_call(
      _stats_body,
      grid=(_GRID,),
      in_specs=[
          pl.BlockSpec((_BLK, _H), lambda i: (i, 0)),
          pl.BlockSpec((_BLK, 8), lambda i: (i, 0)),
          pl.BlockSpec((1, _H), lambda i: (0, 0)),
      ],
      out_specs=pl.BlockSpec((8, _H), lambda i: (0, 0)),
      out_shape=jax.ShapeDtypeStruct((8, _H), jnp.float32),
  )(a, dinv, bg)


def _bn_from_sums(s_ref, gam_ref, bet_ref):
  m = s_ref[0:1, :] * (1.0 / _N)
  var = s_ref[1:2, :] * (1.0 / _N) - m * m
  scale = lax.rsqrt(var + _EPS) * gam_ref[...]
  shift = bet_ref[...] - m * scale
  return scale, shift


def _mid_body(a_ref, dinv_ref, s_ref, bg_ref, gam_ref, bet_ref,
              wn_ref, g_ref):
  dinv = dinv_ref[:, 0:1]
  scale, shift = _bn_from_sums(s_ref, gam_ref, bet_ref)
  y = a_ref[...] * dinv + bg_ref[...]
  h = jnp.maximum(y * scale + shift, 0.0)
  g_ref[...] = _dotT(h, wn_ref[...]) * dinv


def _mid(a, dinv, s, bg, gam, bet, wn):
  return pl.pallas_call(
      _mid_body,
      grid=(_GRID,),
      in_specs=[
          pl.BlockSpec((_BLK, _H), lambda i: (i, 0)),
          pl.BlockSpec((_BLK, 8), lambda i: (i, 0)),
          pl.BlockSpec((8, _H), lambda i: (0, 0)),
          pl.BlockSpec((1, _H), lambda i: (0, 0)),
          pl.BlockSpec((1, _H), lambda i: (0, 0)),
          pl.BlockSpec((1, _H), lambda i: (0, 0)),
          pl.BlockSpec((_H, _H), lambda i: (0, 0)),
      ],
      out_specs=pl.BlockSpec((_BLK, _H), lambda i: (i, 0)),
      out_shape=jax.ShapeDtypeStruct((_ND, _H), jnp.float32),
  )(a, dinv, s, bg, gam, bet, wn)


def _fin_body(a_ref, dinv_ref, s_ref, bg_ref, gam_ref, bet_ref,
              wc1_ref, bc1_ref, wc2_ref, bc2_ref, o_ref):
  dinv = dinv_ref[:, 0:1]
  scale, shift = _bn_from_sums(s_ref, gam_ref, bet_ref)
  y = a_ref[...] * dinv + bg_ref[...]
  h = y * scale + shift
  t = jnp.maximum(_dotT(h, wc1_ref[...]) + bc1_ref[...], 0.0)
  o_ref[...] = _dotT(t, wc2_ref[...]) + bc2_ref[...]


def _fin(a, dinv, s, bg, gam, bet, wc1, bc1, wc2, bc2):
  hc = wc1.shape[0]
  nc = wc2.shape[0]
  return pl.pallas_call(
      _fin_body,
      grid=(_GRID,),
      in_specs=[
          pl.BlockSpec((_BLK, _H), lambda i: (i, 0)),
          pl.BlockSpec((_BLK, 8), lambda i: (i, 0)),
          pl.BlockSpec((8, _H), lambda i: (0, 0)),
          pl.BlockSpec((1, _H), lambda i: (0, 0)),
          pl.BlockSpec((1, _H), lambda i: (0, 0)),
          pl.BlockSpec((1, _H), lambda i: (0, 0)),
          pl.BlockSpec((hc, _H), lambda i: (0, 0)),
          pl.BlockSpec((1, hc), lambda i: (0, 0)),
          pl.BlockSpec((nc, hc), lambda i: (0, 0)),
          pl.BlockSpec((1, nc), lambda i: (0, 0)),
      ],
      out_specs=pl.BlockSpec((_BLK, nc), lambda i: (i, 0)),
      out_shape=jax.ShapeDtypeStruct((_N, nc), jnp.float32),
  )(a, dinv, s, bg, gam, bet, wc1, bc1, wc2, bc2)


def kernel(x, edge_index, Wp, bp, Wg0, bg0, Wg1, bg1, Wg2, bg2,
           gamma0, beta0, gamma1, beta1, gamma2, beta2,
           Wc1, bc1, Wc2, bc2):
  src = edge_index[0]
  dst = edge_index[1]
  pad = _EP - _E
  src2d = jnp.concatenate(
      [src, jnp.zeros((pad,), jnp.int32)]).reshape(_NCHUNK, _CW)
  dst2d = jnp.concatenate(
      [dst, jnp.full((pad,), _TRASH, jnp.int32)]).reshape(_NCHUNK, _CW)

  ones_h = jnp.ones((_CW, 8), jnp.float32)
  zeros_h = jnp.zeros((_ZR, 8), jnp.float32)

  bp_ = bp.reshape(1, _H)
  bg0_ = bg0.reshape(1, _H)
  bg1_ = bg1.reshape(1, _H)
  bg2_ = bg2.reshape(1, _H)
  ga0_ = gamma0.reshape(1, _H)
  be0_ = beta0.reshape(1, _H)
  ga1_ = gamma1.reshape(1, _H)
  be1_ = beta1.reshape(1, _H)
  ga2_ = gamma2.reshape(1, _H)
  be2_ = beta2.reshape(1, _H)
  bc1_ = bc1.reshape(1, -1)
  bc2_ = bc2.reshape(1, -1)

  d0, d1 = _deg_counts(dst2d, ones_h, zeros_h)

  t, dinv = _proj(x, d0, d1, Wp, bp_, Wg0)

  a = _spmm(t, src2d, dst2d)
  s = _stats(a, dinv, bg0_)
  t = _mid(a, dinv, s, bg0_, ga0_, be0_, Wg1)

  a = _spmm(t, src2d, dst2d)
  s = _stats(a, dinv, bg1_)
  t = _mid(a, dinv, s, bg1_, ga1_, be1_, Wg2)

  a = _spmm(t, src2d, dst2d)
  s = _stats(a, dinv, bg2_)
  return _fin(a, dinv, s, bg2_, ga2_, be2_, Wc1, bc1_, Wc2, bc2_)


# speedup vs baseline: 1.6958x; 1.0701x over previous
import functools

import jax
import jax.numpy as jnp
from jax import lax
from jax.experimental import pallas as pl
from jax.experimental.pallas import tpu as pltpu
from jax.experimental.pallas import tpu_sc as plsc

_EPS = 1e-5

_N = 50000
_E = 800000
_H = 64
_HH = 32
_HQ = 16
_BLK = 5000
_GRID = _N // _BLK

_CW = 128
_NCHUNK = 6400
_EP = _NCHUNK * _CW
_CPT = _NCHUNK // 16
_DCPT = _NCHUNK // 32
_SUPC = 80
_SUPS = _CPT // _SUPC
_ND = 50048
_NR = _ND
_TRASH = 50000
_RPT = _ND // 16
_ZR = _NR // 16


def _dotT(a, w):
  return lax.dot_general(a, w, (((1,), (1,)), ((), ())),
                         preferred_element_type=jnp.float32)


def _deg_body(dst2d, ones_h, zeros_h, out0, out1, acc, dall, ones_v):
  c = lax.axis_index("c")
  s = lax.axis_index("s")
  wid = c * 16 + s
  pltpu.sync_copy(ones_h, ones_v)
  pltpu.sync_copy(dst2d.at[pl.ds(wid * _DCPT, _DCPT)], dall)
  pltpu.sync_copy(zeros_h, acc.at[pl.ds(s * _ZR, _ZR)])
  plsc.subcore_barrier()

  @pl.loop(0, _DCPT)
  def _(j):
    pltpu.sync_copy(ones_v, acc.at[dall.at[j]], add=True)

  plsc.subcore_barrier()

  @pl.when(c == 0)
  def _():
    pltpu.sync_copy(acc.at[pl.ds(s * _RPT, _RPT)],
                    out0.at[pl.ds(s * _RPT, _RPT)])

  @pl.when(c == 1)
  def _():
    pltpu.sync_copy(acc.at[pl.ds(s * _RPT, _RPT)],
                    out1.at[pl.ds(s * _RPT, _RPT)])


def _deg_counts(dst2d, ones_h, zeros_h):
  mesh = plsc.VectorSubcoreMesh(core_axis_name="c", subcore_axis_name="s")
  f = pl.kernel(
      _deg_body,
      compiler_params=pltpu.CompilerParams(use_tc_tiling_on_sc=False),
      out_type=[jax.ShapeDtypeStruct((_ND, 8), jnp.float32),
                jax.ShapeDtypeStruct((_ND, 8), jnp.float32)],
      mesh=mesh,
      scratch_types=[
          pltpu.VMEM_SHARED((_NR, 8), jnp.float32),
          pltpu.VMEM((_DCPT, _CW), jnp.int32),
          pltpu.VMEM((_CW, 8), jnp.float32),
      ],
  )
  return f(dst2d, ones_h, zeros_h)


def _spmm_body(tcat, src2d, dst2d, zq, ocat, tblS, accS, sall, dall,
               rowa, rowb, gsa, gsb, ssa, ssb):
  c = lax.axis_index("c")
  s = lax.axis_index("s")

  def run(q):
    rb = s * _RPT
    cb = pl.multiple_of(16 * (2 * c + q), 16)
    pltpu.sync_copy(tcat.at[pl.ds(rb, _RPT), pl.ds(cb, _HQ)],
                    tblS.at[pl.ds(rb, _RPT)])
    pltpu.sync_copy(zq, accS.at[pl.ds(rb, _RPT)])
    plsc.subcore_barrier()

    @pl.loop(0, _SUPS)
    def _(u):
      base = s * _CPT + u * _SUPC
      pltpu.sync_copy(src2d.at[pl.ds(base, _SUPC)], sall)
      pltpu.sync_copy(dst2d.at[pl.ds(base, _SUPC)], dall)
      pltpu.async_copy(tblS.at[sall.at[0]], rowa, gsa)

      @pl.loop(0, _SUPC, step=2)
      def _(j):
        pltpu.make_async_copy(tblS.at[sall.at[j]], rowa, gsa).wait()

        @pl.when(j > 0)
        def _():
          pltpu.make_async_copy(rowb, accS.at[dall.at[j]], ssb).wait()

        pltpu.async_copy(tblS.at[sall.at[j + 1]], rowb, gsb)
        pltpu.async_copy(rowa, accS.at[dall.at[j]], ssa, add=True)
        pltpu.make_async_copy(tblS.at[sall.at[j + 1]], rowb, gsb).wait()
        pltpu.make_async_copy(rowa, accS.at[dall.at[j]], ssa).wait()

        @pl.when(j + 2 < _SUPC)
        def _():
          pltpu.async_copy(tblS.at[sall.at[j + 2]], rowa, gsa)

        pltpu.async_copy(rowb, accS.at[dall.at[j + 1]], ssb, add=True)

      pltpu.make_async_copy(rowb, accS.at[dall.at[0]], ssb).wait()

    plsc.subcore_barrier()
    pltpu.sync_copy(accS.at[pl.ds(rb, _RPT)],
                    ocat.at[pl.ds(rb, _RPT), pl.ds(cb, _HQ)])

  run(0)
  run(1)


def _spmm(tcat, src2w, dst2w, zq):
  mesh = plsc.VectorSubcoreMesh(core_axis_name="c", subcore_axis_name="s")
  f = pl.kernel(
      _spmm_body,
      compiler_params=pltpu.CompilerParams(use_tc_tiling_on_sc=False),
      out_type=jax.ShapeDtypeStruct((_ND, _H), jnp.float32),
      mesh=mesh,
      scratch_types=[
          pltpu.VMEM_SHARED((_NR, _HQ), jnp.float32),
          pltpu.VMEM_SHARED((_NR, _HQ), jnp.float32),
          pltpu.VMEM((_SUPC, _CW), jnp.int32),
          pltpu.VMEM((_SUPC, _CW), jnp.int32),
          pltpu.VMEM((_CW, _HQ), jnp.float32),
          pltpu.VMEM((_CW, _HQ), jnp.float32),
          pltpu.SemaphoreType.DMA,
          pltpu.SemaphoreType.DMA,
          pltpu.SemaphoreType.DMA,
          pltpu.SemaphoreType.DMA,
      ],
  )
  return f(tcat, src2w, dst2w, zq)


def _proj_body(x_ref, d0_ref, d1_ref, wp_ref, bp_ref, wg_ref,
               g_ref, dinv_ref):
  deg = d0_ref[:, 0:1] + d1_ref[:, 0:1] + 1.0
  dinv = lax.rsqrt(deg)
  h = jnp.maximum(_dotT(x_ref[...], wp_ref[...]) + bp_ref[...], 0.0)
  g_ref[...] = _dotT(h, wg_ref[...]) * dinv
  dinv_ref[...] = jnp.broadcast_to(dinv, dinv_ref.shape)


def _proj(x, d0, d1, wp, bp, wg):
  d = x.shape[1]
  return pl.pallas_call(
      _proj_body,
      grid=(_GRID,),
      in_specs=[
          pl.BlockSpec((_BLK, d), lambda i: (i, 0)),
          pl.BlockSpec((_BLK, 8), lambda i: (i, 0)),
          pl.BlockSpec((_BLK, 8), lambda i: (i, 0)),
          pl.BlockSpec((_H, d), lambda i: (0, 0)),
          pl.BlockSpec((1, _H), lambda i: (0, 0)),
          pl.BlockSpec((_H, _H), lambda i: (0, 0)),
      ],
      out_specs=[
          pl.BlockSpec((_BLK, _H), lambda i: (i, 0)),
          pl.BlockSpec((_BLK, 8), lambda i: (i, 0)),
      ],
      out_shape=[
          jax.ShapeDtypeStruct((_ND, _H), jnp.float32),
          jax.ShapeDtypeStruct((_N, 8), jnp.float32),
      ],
  )(x, d0, d1, wp, bp, wg)


def _stats_body(a_ref, t_ref, dinv_ref, bg_ref, s_ref):
  @pl.when(pl.program_id(0) == 0)
  def _():
    s_ref[...] = jnp.zeros_like(s_ref)

  dinv = dinv_ref[:, 0:1]
  y = (a_ref[...] + t_ref[...]) * dinv + bg_ref[...]
  s1 = jnp.sum(y, axis=0, keepdims=True)
  s2 = jnp.sum(y * y, axis=0, keepdims=True)
  s_ref[...] += jnp.concatenate([s1, s2, jnp.zeros((6, _H), jnp.float32)], 0)


def _stats(a, t, dinv, bg):
  return pl.pallas_call(
      _stats_body,
      grid=(_GRID,),
      in_specs=[
          pl.BlockSpec((_BLK, _H), lambda i: (i, 0)),
          pl.BlockSpec((_BLK, _H), lambda i: (i, 0)),
          pl.BlockSpec((_BLK, 8), lambda i: (i, 0)),
          pl.BlockSpec((1, _H), lambda i: (0, 0)),
      ],
      out_specs=pl.BlockSpec((8, _H), lambda i: (0, 0)),
      out_shape=jax.ShapeDtypeStruct((8, _H), jnp.float32),
  )(a, t, dinv, bg)


def _bn_from_sums(s_ref, gam_ref, bet_ref):
  m = s_ref[0:1, :] * (1.0 / _N)
  var = s_ref[1:2, :] * (1.0 / _N) - m * m
  scale = lax.rsqrt(var + _EPS) * gam_ref[...]
  shift = bet_ref[...] - m * scale
  return scale, shift


def _mid_body(a_ref, t_ref, dinv_ref, s_ref, bg_ref, gam_ref, bet_ref,
              wn_ref, g_ref):
  dinv = dinv_ref[:, 0:1]
  scale, shift = _bn_from_sums(s_ref, gam_ref, bet_ref)
  y = (a_ref[...] + t_ref[...]) * dinv + bg_ref[...]
  h = jnp.maximum(y * scale + shift, 0.0)
  g_ref[...] = _dotT(h, wn_ref[...]) * dinv


def _mid(a, t, dinv, s, bg, gam, bet, wn):
  return pl.pallas_call(
      _mid_body,
      grid=(_GRID,),
      in_specs=[
          pl.BlockSpec((_BLK, _H), lambda i: (i, 0)),
          pl.BlockSpec((_BLK, _H), lambda i: (i, 0)),
          pl.BlockSpec((_BLK, 8), lambda i: (i, 0)),
          pl.BlockSpec((8, _H), lambda i: (0, 0)),
          pl.BlockSpec((1, _H), lambda i: (0, 0)),
          pl.BlockSpec((1, _H), lambda i: (0, 0)),
          pl.BlockSpec((1, _H), lambda i: (0, 0)),
          pl.BlockSpec((_H, _H), lambda i: (0, 0)),
      ],
      out_specs=pl.BlockSpec((_BLK, _H), lambda i: (i, 0)),
      out_shape=jax.ShapeDtypeStruct((_ND, _H), jnp.float32),
  )(a, t, dinv, s, bg, gam, bet, wn)


def _fin_body(a_ref, t_ref, dinv_ref, s_ref, bg_ref, gam_ref, bet_ref,
              wc1_ref, bc1_ref, wc2_ref, bc2_ref, o_ref):
  dinv = dinv_ref[:, 0:1]
  scale, shift = _bn_from_sums(s_ref, gam_ref, bet_ref)
  y = (a_ref[...] + t_ref[...]) * dinv + bg_ref[...]
  h = y * scale + shift
  t = jnp.maximum(_dotT(h, wc1_ref[...]) + bc1_ref[...], 0.0)
  o_ref[...] = _dotT(t, wc2_ref[...]) + bc2_ref[...]


def _fin(a, t, dinv, s, bg, gam, bet, wc1, bc1, wc2, bc2):
  hc = wc1.shape[0]
  nc = wc2.shape[0]
  return pl.pallas_call(
      _fin_body,
      grid=(_GRID,),
      in_specs=[
          pl.BlockSpec((_BLK, _H), lambda i: (i, 0)),
          pl.BlockSpec((_BLK, _H), lambda i: (i, 0)),
          pl.BlockSpec((_BLK, 8), lambda i: (i, 0)),
          pl.BlockSpec((8, _H), lambda i: (0, 0)),
          pl.BlockSpec((1, _H), lambda i: (0, 0)),
          pl.BlockSpec((1, _H), lambda i: (0, 0)),
          pl.BlockSpec((1, _H), lambda i: (0, 0)),
          pl.BlockSpec((hc, _H), lambda i: (0, 0)),
          pl.BlockSpec((1, hc), lambda i: (0, 0)),
          pl.BlockSpec((nc, hc), lambda i: (0, 0)),
          pl.BlockSpec((1, nc), lambda i: (0, 0)),
      ],
      out_specs=pl.BlockSpec((_BLK, nc), lambda i: (i, 0)),
      out_shape=jax.ShapeDtypeStruct((_N, nc), jnp.float32),
  )(a, t, dinv, s, bg, gam, bet, wc1, bc1, wc2, bc2)


def kernel(x, edge_index, Wp, bp, Wg0, bg0, Wg1, bg1, Wg2, bg2,
           gamma0, beta0, gamma1, beta1, gamma2, beta2,
           Wc1, bc1, Wc2, bc2):
  src = edge_index[0]
  dst = edge_index[1]
  pad = _EP - _E
  src2d = jnp.concatenate(
      [src, jnp.zeros((pad,), jnp.int32)]).reshape(_NCHUNK, _CW)
  dst2d = jnp.concatenate(
      [dst, jnp.full((pad,), _TRASH, jnp.int32)]).reshape(_NCHUNK, _CW)

  ones_h = jnp.ones((_CW, 8), jnp.float32)
  zeros_h = jnp.zeros((_ZR, 8), jnp.float32)
  zq = jnp.zeros((_RPT, _HQ), jnp.float32)

  bp_ = bp.reshape(1, _H)
  bg0_ = bg0.reshape(1, _H)
  bg1_ = bg1.reshape(1, _H)
  bg2_ = bg2.reshape(1, _H)
  ga0_ = gamma0.reshape(1, _H)
  be0_ = beta0.reshape(1, _H)
  ga1_ = gamma1.reshape(1, _H)
  be1_ = beta1.reshape(1, _H)
  ga2_ = gamma2.reshape(1, _H)
  be2_ = beta2.reshape(1, _H)
  bc1_ = bc1.reshape(1, -1)
  bc2_ = bc2.reshape(1, -1)

  d0, d1 = _deg_counts(dst2d, ones_h, zeros_h)

  t, dinv = _proj(x, d0, d1, Wp, bp_, Wg0)

  a = _spmm(t, src2d, dst2d, zq)
  s = _stats(a, t, dinv, bg0_)
  t = _mid(a, t, dinv, s, bg0_, ga0_, be0_, Wg1)

  a = _spmm(t, src2d, dst2d, zq)
  s = _stats(a, t, dinv, bg1_)
  t = _mid(a, t, dinv, s, bg1_, ga1_, be1_, Wg2)

  a = _spmm(t, src2d, dst2d, zq)
  s = _stats(a, t, dinv, bg2_)
  return _fin(a, t, dinv, s, bg2_, ga2_, be2_, Wc1, bc1_, Wc2, bc2_)


# speedup vs baseline: 1.6959x; 1.0000x over previous
import jax
import jax.numpy as jnp
from jax import lax
from jax.experimental import pallas as pl
from jax.experimental.pallas import tpu as pltpu
from jax.experimental.pallas import tpu_sc as plsc

_EPS = 1e-5

_N = 50000
_E = 800000
_H = 64
_HQ = 16
_BLK = 5000
_GRID = _N // _BLK

_CW = 128
_NCHUNK = 6400
_EP = _NCHUNK * _CW
_CPT = _NCHUNK // 16
_DCPT = _NCHUNK // 32
_SUPC = 80
_SUPS = _CPT // _SUPC
_ND = 50048
_NR = _ND
_TRASH = 50000
_RPT = _ND // 16
_ZR = _NR // 16


def _dotT(a, w):
  return lax.dot_general(a, w, (((1,), (1,)), ((), ())),
                         preferred_element_type=jnp.float32)


def _deg_body(dst2d, ones_h, zeros_h, out0, out1, acc, dall, ones_v):
  c = lax.axis_index("c")
  s = lax.axis_index("s")
  wid = c * 16 + s
  pltpu.sync_copy(ones_h, ones_v)
  pltpu.sync_copy(dst2d.at[pl.ds(wid * _DCPT, _DCPT)], dall)
  pltpu.sync_copy(zeros_h, acc.at[pl.ds(s * _ZR, _ZR)])
  plsc.subcore_barrier()

  @pl.loop(0, _DCPT)
  def _(j):
    pltpu.sync_copy(ones_v, acc.at[dall.at[j]], add=True)

  plsc.subcore_barrier()

  @pl.when(c == 0)
  def _():
    pltpu.sync_copy(acc.at[pl.ds(s * _RPT, _RPT)],
                    out0.at[pl.ds(s * _RPT, _RPT)])

  @pl.when(c == 1)
  def _():
    pltpu.sync_copy(acc.at[pl.ds(s * _RPT, _RPT)],
                    out1.at[pl.ds(s * _RPT, _RPT)])


def _deg_counts(dst2d, ones_h, zeros_h):
  mesh = plsc.VectorSubcoreMesh(core_axis_name="c", subcore_axis_name="s")
  f = pl.kernel(
      _deg_body,
      compiler_params=pltpu.CompilerParams(use_tc_tiling_on_sc=False),
      out_type=[jax.ShapeDtypeStruct((_ND, 8), jnp.float32),
                jax.ShapeDtypeStruct((_ND, 8), jnp.float32)],
      mesh=mesh,
      scratch_types=[
          pltpu.VMEM_SHARED((_NR, 8), jnp.float32),
          pltpu.VMEM((_DCPT, _CW), jnp.int32),
          pltpu.VMEM((_CW, 8), jnp.float32),
      ],
  )
  return f(dst2d, ones_h, zeros_h)


def _spmm_body(tcat, src2d, dst2d, zq, ocat, tblS, accS, sall, dall,
               rowa, rowb, gsa, gsb, ssa, ssb):
  c = lax.axis_index("c")
  s = lax.axis_index("s")

  def run(q):
    rb = s * _RPT
    cb = pl.multiple_of(16 * (2 * c + q), 16)
    pltpu.sync_copy(tcat.at[pl.ds(rb, _RPT), pl.ds(cb, _HQ)],
                    tblS.at[pl.ds(rb, _RPT)])
    pltpu.sync_copy(zq, accS.at[pl.ds(rb, _RPT)])
    plsc.subcore_barrier()

    @pl.loop(0, _SUPS)
    def _(u):
      base = s * _CPT + u * _SUPC
      pltpu.sync_copy(src2d.at[pl.ds(base, _SUPC)], sall)
      pltpu.sync_copy(dst2d.at[pl.ds(base, _SUPC)], dall)
      pltpu.async_copy(tblS.at[sall.at[0]], rowa, gsa)

      @pl.loop(0, _SUPC, step=2)
      def _(j):
        pltpu.make_async_copy(tblS.at[sall.at[j]], rowa, gsa).wait()

        @pl.when(j > 0)
        def _():
          pltpu.make_async_copy(rowb, accS.at[dall.at[j]], ssb).wait()

        pltpu.async_copy(tblS.at[sall.at[j + 1]], rowb, gsb)
        pltpu.async_copy(rowa, accS.at[dall.at[j]], ssa, add=True)
        pltpu.make_async_copy(tblS.at[sall.at[j + 1]], rowb, gsb).wait()
        pltpu.make_async_copy(rowa, accS.at[dall.at[j]], ssa).wait()

        @pl.when(j + 2 < _SUPC)
        def _():
          pltpu.async_copy(tblS.at[sall.at[j + 2]], rowa, gsa)

        pltpu.async_copy(rowb, accS.at[dall.at[j + 1]], ssb, add=True)

      pltpu.make_async_copy(rowb, accS.at[dall.at[0]], ssb).wait()

    plsc.subcore_barrier()
    pltpu.sync_copy(accS.at[pl.ds(rb, _RPT)],
                    ocat.at[pl.ds(rb, _RPT), pl.ds(cb, _HQ)])

  run(0)
  run(1)


def _spmm(tcat, src2w, dst2w, zq):
  mesh = plsc.VectorSubcoreMesh(core_axis_name="c", subcore_axis_name="s")
  f = pl.kernel(
      _spmm_body,
      compiler_params=pltpu.CompilerParams(use_tc_tiling_on_sc=False),
      out_type=jax.ShapeDtypeStruct((_ND, _H), jnp.float32),
      mesh=mesh,
      scratch_types=[
          pltpu.VMEM_SHARED((_NR, _HQ), jnp.float32),
          pltpu.VMEM_SHARED((_NR, _HQ), jnp.float32),
          pltpu.VMEM((_SUPC, _CW), jnp.int32),
          pltpu.VMEM((_SUPC, _CW), jnp.int32),
          pltpu.VMEM((_CW, _HQ), jnp.float32),
          pltpu.VMEM((_CW, _HQ), jnp.float32),
          pltpu.SemaphoreType.DMA,
          pltpu.SemaphoreType.DMA,
          pltpu.SemaphoreType.DMA,
          pltpu.SemaphoreType.DMA,
      ],
  )
  return f(tcat, src2w, dst2w, zq)


def _proj_body(x_ref, d0_ref, d1_ref, wp_ref, bp_ref, wg_ref,
               g_ref, dinv_ref):
  deg = d0_ref[:, 0:1] + d1_ref[:, 0:1] + 1.0
  dinv = lax.rsqrt(deg)
  h = jnp.maximum(_dotT(x_ref[...], wp_ref[...]) + bp_ref[...], 0.0)
  g_ref[...] = _dotT(h, wg_ref[...]) * dinv
  dinv_ref[...] = jnp.broadcast_to(dinv, dinv_ref.shape)


def _proj(x, d0, d1, wp, bp, wg):
  d = x.shape[1]
  return pl.pallas_call(
      _proj_body,
      grid=(_GRID,),
      in_specs=[
          pl.BlockSpec((_BLK, d), lambda i: (i, 0)),
          pl.BlockSpec((_BLK, 8), lambda i: (i, 0)),
          pl.BlockSpec((_BLK, 8), lambda i: (i, 0)),
          pl.BlockSpec((_H, d), lambda i: (0, 0)),
          pl.BlockSpec((1, _H), lambda i: (0, 0)),
          pl.BlockSpec((_H, _H), lambda i: (0, 0)),
      ],
      out_specs=[
          pl.BlockSpec((_BLK, _H), lambda i: (i, 0)),
          pl.BlockSpec((_BLK, 8), lambda i: (i, 0)),
      ],
      out_shape=[
          jax.ShapeDtypeStruct((_ND, _H), jnp.float32),
          jax.ShapeDtypeStruct((_N, 8), jnp.float32),
      ],
  )(x, d0, d1, wp, bp, wg)


def _stats_body(a_ref, t_ref, dinv_ref, bg_ref, s_ref):
  @pl.when(pl.program_id(0) == 0)
  def _():
    s_ref[...] = jnp.zeros_like(s_ref)

  dinv = dinv_ref[:, 0:1]
  y = (a_ref[...] + t_ref[...]) * dinv + bg_ref[...]
  s1 = jnp.sum(y, axis=0, keepdims=True)
  s2 = jnp.sum(y * y, axis=0, keepdims=True)
  s_ref[...] += jnp.concatenate([s1, s2, jnp.zeros((6, _H), jnp.float32)], 0)


def _stats(a, t, dinv, bg):
  return pl.pallas_call(
      _stats_body,
      grid=(_GRID,),
      in_specs=[
          pl.BlockSpec((_BLK, _H), lambda i: (i, 0)),
          pl.BlockSpec((_BLK, _H), lambda i: (i, 0)),
          pl.BlockSpec((_BLK, 8), lambda i: (i, 0)),
          pl.BlockSpec((1, _H), lambda i: (0, 0)),
      ],
      out_specs=pl.BlockSpec((8, _H), lambda i: (0, 0)),
      out_shape=jax.ShapeDtypeStruct((8, _H), jnp.float32),
  )(a, t, dinv, bg)


def _bn_from_sums(s_ref, gam_ref, bet_ref):
  m = s_ref[0:1, :] * (1.0 / _N)
  var = s_ref[1:2, :] * (1.0 / _N) - m * m
  scale = lax.rsqrt(var + _EPS) * gam_ref[...]
  shift = bet_ref[...] - m * scale
  return scale, shift


def _mid_body(a_ref, t_ref, dinv_ref, s_ref, bg_ref, gam_ref, bet_ref,
              wn_ref, g_ref):
  dinv = dinv_ref[:, 0:1]
  scale, shift = _bn_from_sums(s_ref, gam_ref, bet_ref)
  y = (a_ref[...] + t_ref[...]) * dinv + bg_ref[...]
  h = jnp.maximum(y * scale + shift, 0.0)
  g_ref[...] = _dotT(h, wn_ref[...]) * dinv


def _mid(a, t, dinv, s, bg, gam, bet, wn):
  return pl.pallas_call(
      _mid_body,
      grid=(_GRID,),
      in_specs=[
          pl.BlockSpec((_BLK, _H), lambda i: (i, 0)),
          pl.BlockSpec((_BLK, _H), lambda i: (i, 0)),
          pl.BlockSpec((_BLK, 8), lambda i: (i, 0)),
          pl.BlockSpec((8, _H), lambda i: (0, 0)),
          pl.BlockSpec((1, _H), lambda i: (0, 0)),
          pl.BlockSpec((1, _H), lambda i: (0, 0)),
          pl.BlockSpec((1, _H), lambda i: (0, 0)),
          pl.BlockSpec((_H, _H), lambda i: (0, 0)),
      ],
      out_specs=pl.BlockSpec((_BLK, _H), lambda i: (i, 0)),
      out_shape=jax.ShapeDtypeStruct((_ND, _H), jnp.float32),
  )(a, t, dinv, s, bg, gam, bet, wn)


def _fin_body(a_ref, t_ref, dinv_ref, s_ref, bg_ref, gam_ref, bet_ref,
              wc1_ref, bc1_ref, wc2_ref, bc2_ref, o_ref):
  dinv = dinv_ref[:, 0:1]
  scale, shift = _bn_from_sums(s_ref, gam_ref, bet_ref)
  y = (a_ref[...] + t_ref[...]) * dinv + bg_ref[...]
  h = y * scale + shift
  t = jnp.maximum(_dotT(h, wc1_ref[...]) + bc1_ref[...], 0.0)
  o_ref[...] = _dotT(t, wc2_ref[...]) + bc2_ref[...]


def _fin(a, t, dinv, s, bg, gam, bet, wc1, bc1, wc2, bc2):
  hc = wc1.shape[0]
  nc = wc2.shape[0]
  return pl.pallas_call(
      _fin_body,
      grid=(_GRID,),
      in_specs=[
          pl.BlockSpec((_BLK, _H), lambda i: (i, 0)),
          pl.BlockSpec((_BLK, _H), lambda i: (i, 0)),
          pl.BlockSpec((_BLK, 8), lambda i: (i, 0)),
          pl.BlockSpec((8, _H), lambda i: (0, 0)),
          pl.BlockSpec((1, _H), lambda i: (0, 0)),
          pl.BlockSpec((1, _H), lambda i: (0, 0)),
          pl.BlockSpec((1, _H), lambda i: (0, 0)),
          pl.BlockSpec((hc, _H), lambda i: (0, 0)),
          pl.BlockSpec((1, hc), lambda i: (0, 0)),
          pl.BlockSpec((nc, hc), lambda i: (0, 0)),
          pl.BlockSpec((1, nc), lambda i: (0, 0)),
      ],
      out_specs=pl.BlockSpec((_BLK, nc), lambda i: (i, 0)),
      out_shape=jax.ShapeDtypeStruct((_N, nc), jnp.float32),
  )(a, t, dinv, s, bg, gam, bet, wc1, bc1, wc2, bc2)


def kernel(x, edge_index, Wp, bp, Wg0, bg0, Wg1, bg1, Wg2, bg2,
           gamma0, beta0, gamma1, beta1, gamma2, beta2,
           Wc1, bc1, Wc2, bc2):
  src = edge_index[0]
  dst = edge_index[1]
  pad = _EP - _E
  src2d = jnp.concatenate(
      [src, jnp.zeros((pad,), jnp.int32)]).reshape(_NCHUNK, _CW)
  dst2d = jnp.concatenate(
      [dst, jnp.full((pad,), _TRASH, jnp.int32)]).reshape(_NCHUNK, _CW)

  ones_h = jnp.ones((_CW, 8), jnp.float32)
  zeros_h = jnp.zeros((_ZR, 8), jnp.float32)
  zq = jnp.zeros((_RPT, _HQ), jnp.float32)

  bp_ = bp.reshape(1, _H)
  bg0_ = bg0.reshape(1, _H)
  bg1_ = bg1.reshape(1, _H)
  bg2_ = bg2.reshape(1, _H)
  ga0_ = gamma0.reshape(1, _H)
  be0_ = beta0.reshape(1, _H)
  ga1_ = gamma1.reshape(1, _H)
  be1_ = beta1.reshape(1, _H)
  ga2_ = gamma2.reshape(1, _H)
  be2_ = beta2.reshape(1, _H)
  bc1_ = bc1.reshape(1, -1)
  bc2_ = bc2.reshape(1, -1)

  d0, d1 = _deg_counts(dst2d, ones_h, zeros_h)

  t, dinv = _proj(x, d0, d1, Wp, bp_, Wg0)

  a = _spmm(t, src2d, dst2d, zq)
  s = _stats(a, t, dinv, bg0_)
  t = _mid(a, t, dinv, s, bg0_, ga0_, be0_, Wg1)

  a = _spmm(t, src2d, dst2d, zq)
  s = _stats(a, t, dinv, bg1_)
  t = _mid(a, t, dinv, s, bg1_, ga1_, be1_, Wg2)

  a = _spmm(t, src2d, dst2d, zq)
  s = _stats(a, t, dinv, bg2_)
  return _fin(a, t, dinv, s, bg2_, ga2_, be2_, Wc1, bc1_, Wc2, bc2_)
